# baseline (speedup 1.0000x reference)
import functools

import jax
import jax.numpy as jnp
from jax import lax
from jax.experimental import pallas as pl
from jax.experimental.pallas import tpu as pltpu
from jax.experimental.pallas import tpu_sc as plsc

B = 16384
D = 64
NC = 2
NS = 16
NW = NC * NS
BPW = B // NW
WAVE = 128
NWAVE = BPW // WAVE
NBROW = 7813


def _body(u_idx_hbm, i_idx_hbm, ut2_hbm, ubp_hbm, it2_hbm, ibp_hbm,
          out_hbm,
          iv_u, iv_i, r2u, r2i, r3u, r3i,
          gbu, gbi, gbub, gbib, out_v, sem):
    wid = lax.axis_index("s") * NC + lax.axis_index("c")
    base = wid * BPW

    pltpu.sync_copy(u_idx_hbm.at[pl.ds(base, BPW)], iv_u)
    pltpu.sync_copy(i_idx_hbm.at[pl.ds(base, BPW)], iv_i)

    for k in range(BPW // 16):
        w, off = k // (WAVE // 16), (k % (WAVE // 16)) * 16
        sl = pl.ds(k * 16, 16)
        dsl = pl.ds(off, 16)
        u16 = iv_u[sl]
        i16 = iv_i[sl]
        r2u[w, dsl] = lax.shift_right_logical(u16, 1)
        r2i[w, dsl] = lax.shift_right_logical(i16, 1)
        r3u[w, dsl] = lax.shift_right_logical(u16, 7)
        r3i[w, dsl] = lax.shift_right_logical(i16, 7)

    lanes = lax.iota(jnp.int32, 16)

    for w in range(NWAVE):
        cps = (
            pltpu.make_async_copy(ut2_hbm.at[r2u.at[w]], gbu, sem),
            pltpu.make_async_copy(it2_hbm.at[r2i.at[w]], gbi, sem),
            pltpu.make_async_copy(ubp_hbm.at[r3u.at[w]], gbub, sem),
            pltpu.make_async_copy(ibp_hbm.at[r3i.at[w]], gbib, sem),
        )
        for cp in cps:
            cp.start()
        for cp in cps:
            cp.wait()

        def grp(g, _, w=w):
            sl = pl.ds(w * WAVE + g * 16, 16)
            lsl = pl.ds(g * 16, 16)
            u16 = iv_u[sl]
            i16 = iv_i[sl]
            rr16 = g * 16 + lanes
            offu = (u16 & 1) * D
            offi = (i16 & 1) * D

            def col(c, acc):
                vu = plsc.load_gather(gbu, [rr16, offu + c])
                vi = plsc.load_gather(gbi, [rr16, offi + c])
                return acc + vu * vi

            acc0 = (plsc.load_gather(gbub, [rr16, u16 & 127])
                    + plsc.load_gather(gbib, [rr16, i16 & 127]))
            x = lax.fori_loop(0, D, col, acc0)
            out_v[sl] = 1.0 / (1.0 + jnp.exp(-x))
            return 0

        lax.fori_loop(0, WAVE // 16, grp, 0)

    pltpu.sync_copy(out_v, out_hbm.at[pl.ds(base, BPW)])


@functools.partial(jax.jit, static_argnames=())
def _run(u_idx, i_idx, ut2, ubp, it2, ibp):
    mesh = plsc.VectorSubcoreMesh(core_axis_name="c", subcore_axis_name="s",
                                  num_cores=NC, num_subcores=NS)
    f = pl.kernel(
        _body,
        out_type=jax.ShapeDtypeStruct((B,), jnp.float32),
        mesh=mesh,
        compiler_params=pltpu.CompilerParams(needs_layout_passes=False,
                                             use_tc_tiling_on_sc=True),
        scratch_types=[
            pltpu.VMEM((BPW,), jnp.int32),
            pltpu.VMEM((BPW,), jnp.int32),
            pltpu.VMEM((NWAVE, WAVE), jnp.int32),
            pltpu.VMEM((NWAVE, WAVE), jnp.int32),
            pltpu.VMEM((NWAVE, WAVE), jnp.int32),
            pltpu.VMEM((NWAVE, WAVE), jnp.int32),
            pltpu.VMEM((WAVE, 128), jnp.float32),
            pltpu.VMEM((WAVE, 128), jnp.float32),
            pltpu.VMEM((WAVE, 128), jnp.float32),
            pltpu.VMEM((WAVE, 128), jnp.float32),
            pltpu.VMEM((BPW,), jnp.float32),
            pltpu.SemaphoreType.DMA,
        ],
    )
    return f(u_idx, i_idx, ut2, ubp, it2, ibp)


_TBLK = 2048


def _tr_body(x_ref, o_ref):
    o_ref[...] = x_ref[...].T


def _to_rows(table):
    n = table.shape[0]
    grid = (n + _TBLK - 1) // _TBLK
    out = pl.pallas_call(
        _tr_body,
        grid=(grid,),
        in_specs=[pl.BlockSpec((D, _TBLK), lambda j: (0, j))],
        out_specs=pl.BlockSpec((_TBLK, D), lambda j: (j, 0)),
        out_shape=jax.ShapeDtypeStruct((n, D), jnp.float32),
    )(table.T)
    return out.reshape(-1, 128)


def kernel(inputs, user_embedding, user_bias, item_embedding, item_bias):
    u_idx = inputs[:, 0]
    i_idx = inputs[:, 1]
    ut2 = _to_rows(user_embedding)
    it2 = _to_rows(item_embedding)
    ubp = jnp.pad(user_bias[:, 0], (0, NBROW * 128 - user_bias.shape[0])
                  ).reshape(NBROW, 128)
    ibp = jnp.pad(item_bias[:, 0], (0, NBROW * 128 - item_bias.shape[0])
                  ).reshape(NBROW, 128)
    out = _run(u_idx, i_idx, ut2, ubp, it2, ibp)
    return out[:, None]
